# baseline (speedup 1.0000x reference)
import jax
import jax.numpy as jnp
from jax.experimental import pallas as pl
from jax.experimental.pallas import tpu as pltpu

_NT = 32768


def _round_up(x, m):
    return ((x + m - 1) // m) * m


def _mlp_fold_kernel(xf_ref, w1a_ref, w2_ref, b2t_ref, out_ref):
    nt = xf_ref.shape[1]
    ones = jnp.ones((1, nt), xf_ref.dtype)
    outs = []
    for p in range(2):
        xa = jnp.concatenate([xf_ref[4 * p:4 * p + 4, :], ones], axis=0)
        ht = jax.lax.dot_general(
            w1a_ref[...], xa, (((0,), (0,)), ((), ())),
            preferred_element_type=jnp.float32)
        ht = jnp.maximum(ht, 0.0)
        ot = jax.lax.dot_general(
            w2_ref[...], ht, (((0,), (0,)), ((), ())),
            preferred_element_type=jnp.float32)
        outs.append(ot + b2t_ref[...])
    o0, o1 = outs
    out_ref[...] = jnp.concatenate(
        [o0[0:1], o1[0:1], o0[1:2], o1[1:2]], axis=0)


def kernel(x, w1, b1, w2_p, b2_p):
    batch, obs = x.shape
    hidden = w1.shape[1]
    n_actions = 2

    b_pad = _round_up(batch, 2 * _NT)
    half = b_pad // 2
    nt = min(_NT, half)
    x_p = x if b_pad == batch else jnp.pad(x, ((0, b_pad - batch), (0, 0)))

    xf = x_p.reshape(2, half, obs).transpose(0, 2, 1).reshape(2 * obs, half)
    w1a = jnp.concatenate([w1, b1], axis=0)
    w2 = w2_p[:, :n_actions]
    b2t = b2_p[:, :n_actions].T

    grid = (half // nt,)
    out4 = pl.pallas_call(
        _mlp_fold_kernel,
        out_shape=jax.ShapeDtypeStruct((2 * n_actions, half), jnp.float32),
        grid=grid,
        in_specs=[
            pl.BlockSpec((2 * obs, nt), lambda i: (0, i)),
            pl.BlockSpec((obs + 1, hidden), lambda i: (0, 0)),
            pl.BlockSpec((hidden, n_actions), lambda i: (0, 0)),
            pl.BlockSpec((n_actions, 1), lambda i: (0, 0)),
        ],
        out_specs=pl.BlockSpec((2 * n_actions, nt), lambda i: (0, i)),
        compiler_params=pltpu.CompilerParams(
            dimension_semantics=("parallel",)),
    )(xf, w1a, w2, b2t)

    out = out4.reshape(n_actions, 2, half).transpose(1, 2, 0)
    return out.reshape(b_pad, n_actions)[:batch]


# speedup vs baseline: 1.5703x; 1.5703x over previous
import jax
import jax.numpy as jnp
from jax.experimental import pallas as pl
from jax.experimental.pallas import tpu as pltpu

_NT = 65536


def _round_up(x, m):
    return ((x + m - 1) // m) * m


def _mlp_t_kernel(xt_ref, w1a_ref, w2_ref, b2t_ref, out_ref):
    xt = xt_ref[...]
    nt = xt.shape[1]
    xa = jnp.concatenate([xt, jnp.ones((1, nt), xt.dtype)], axis=0)
    ht = jax.lax.dot_general(
        w1a_ref[...], xa, (((0,), (0,)), ((), ())),
        preferred_element_type=jnp.float32)
    ht = jnp.maximum(ht, 0.0)
    ot = jax.lax.dot_general(
        w2_ref[...], ht, (((0,), (0,)), ((), ())),
        preferred_element_type=jnp.float32)
    out_ref[...] = ot + b2t_ref[...]


def kernel(x, w1, b1, w2_p, b2_p):
    batch, obs = x.shape
    hidden = w1.shape[1]
    n_actions = 2

    b_pad = _round_up(batch, _NT)
    nt = min(_NT, b_pad)
    x_p = x if b_pad == batch else jnp.pad(x, ((0, b_pad - batch), (0, 0)))

    xt = x_p.T
    w1a = jnp.concatenate([w1, b1], axis=0)
    w2 = w2_p[:, :n_actions]
    b2t = b2_p[:, :n_actions].T

    grid = (b_pad // nt,)
    out_t = pl.pallas_call(
        _mlp_t_kernel,
        out_shape=jax.ShapeDtypeStruct((n_actions, b_pad), jnp.float32),
        grid=grid,
        in_specs=[
            pl.BlockSpec((obs, nt), lambda i: (0, i)),
            pl.BlockSpec((obs + 1, hidden), lambda i: (0, 0)),
            pl.BlockSpec((hidden, n_actions), lambda i: (0, 0)),
            pl.BlockSpec((n_actions, 1), lambda i: (0, 0)),
        ],
        out_specs=pl.BlockSpec((n_actions, nt), lambda i: (0, i)),
        compiler_params=pltpu.CompilerParams(
            dimension_semantics=("parallel",)),
    )(xt, w1a, w2, b2t)

    return out_t[:, :batch].T
